# baseline (speedup 1.0000x reference)
import functools

import jax
import jax.numpy as jnp
from jax import lax
from jax.experimental import pallas as pl
from jax.experimental.pallas import tpu as pltpu
from jax.experimental.pallas import tpu_sc as plsc

B = 16


_SIZES = [300, 300, 30000, 30000, 150, 150, 15000, 15000]
_BASES = [sum(_SIZES[:i]) for i in range(8)]


def _sc_lookup(idx48, tables_cat):
    mesh = plsc.VectorSubcoreMesh(core_axis_name="c", subcore_axis_name="s")
    C3 = 3 * B

    @functools.partial(
        pl.kernel,
        mesh=mesh,
        compiler_params=pltpu.CompilerParams(needs_layout_passes=False, use_tc_tiling_on_sc=True),
        out_type=jax.ShapeDtypeStruct((6, B), jnp.float32),
        scratch_types=[
            pltpu.VMEM((C3,), jnp.int32),
            pltpu.VMEM((8 * C3,), jnp.int32),
            pltpu.VMEM((8 * C3,), jnp.float32),
            pltpu.VMEM((6, B), jnp.float32),
        ],
    )
    def lookup(idx_h, tab_h, wb_out, idx_v, gidx_v, g_v, wb_v):
        wid = lax.axis_index("s") * 2 + lax.axis_index("c")

        @pl.when(wid == 0)
        def _():
            pltpu.sync_copy(idx_h, idx_v)
            cam3 = idx_v[pl.ds(0, B)] * 3
            id3 = idx_v[pl.ds(B, B)] * 3
            for i, base in enumerate(_BASES):
                src = cam3 if i in (0, 1, 4, 5) else id3
                for c in range(3):
                    gidx_v[pl.ds(i * C3 + c * B, B)] = src + (base + c)
            pltpu.sync_copy(tab_h.at[gidx_v], g_v)
            use1 = idx_v[pl.ds(2 * B, B)] == 0
            for c in range(3):
                o = c * B
                wb_v[c, :] = jnp.where(
                    use1,
                    g_v[pl.ds(0 * C3 + o, B)] + g_v[pl.ds(2 * C3 + o, B)],
                    g_v[pl.ds(4 * C3 + o, B)] + g_v[pl.ds(6 * C3 + o, B)])
                wb_v[3 + c, :] = jnp.where(
                    use1,
                    g_v[pl.ds(1 * C3 + o, B)] + g_v[pl.ds(3 * C3 + o, B)],
                    g_v[pl.ds(5 * C3 + o, B)] + g_v[pl.ds(7 * C3 + o, B)])
            pltpu.sync_copy(wb_v, wb_out)

    return lookup(idx48, tables_cat)


NB = 4


def _affine_body(wb_ref, img_ref, out_ref):
    b_i = pl.program_id(0)
    for j in range(NB):
        for c in range(3):
            out_ref[j, c] = (img_ref[j, c] * wb_ref[c, b_i * NB + j]
                             + wb_ref[3 + c, b_i * NB + j])


def _tc_affine(wb, image):
    return pl.pallas_call(
        _affine_body,
        grid=(B // NB,),
        in_specs=[
            pl.BlockSpec(memory_space=pltpu.SMEM),
            pl.BlockSpec((NB, 3, 512, 512), lambda bi: (bi, 0, 0, 0)),
        ],
        out_specs=pl.BlockSpec((NB, 3, 512, 512), lambda bi: (bi, 0, 0, 0)),
        out_shape=jax.ShapeDtypeStruct(image.shape, image.dtype),
        compiler_params=pltpu.CompilerParams(
            dimension_semantics=("parallel",)),
    )(wb, image)


@jax.jit
def kernel(image, camindex, idindex, dataset_type,
           wcam1, bcam1, wident1, bident1,
           wcam2, bcam2, wident2, bident2):
    idx48 = jnp.concatenate([camindex, idindex, dataset_type])
    tables_cat = jnp.concatenate([
        wcam1.reshape(-1), bcam1.reshape(-1),
        wident1.reshape(-1), bident1.reshape(-1),
        wcam2.reshape(-1), bcam2.reshape(-1),
        wident2.reshape(-1), bident2.reshape(-1)])
    wb = _sc_lookup(idx48, tables_cat)
    return _tc_affine(wb, image)


# speedup vs baseline: 1.2438x; 1.2438x over previous
import functools

import jax
import jax.numpy as jnp
from jax import lax
from jax.experimental import pallas as pl
from jax.experimental.pallas import tpu as pltpu
from jax.experimental.pallas import tpu_sc as plsc

B = 16
NB = 4
NIDENT1 = 10000


def _sc_lookup(camindex, idindex, dataset_type,
               wcam1f, bcam1f, wcam2f, bcam2f):
    mesh = plsc.VectorSubcoreMesh(core_axis_name="c", subcore_axis_name="s")

    @functools.partial(
        pl.kernel,
        mesh=mesh,
        compiler_params=pltpu.CompilerParams(needs_layout_passes=False),
        out_type=[jax.ShapeDtypeStruct((6, B), jnp.float32),
                  jax.ShapeDtypeStruct((B,), jnp.int32)],
        scratch_types=[
            pltpu.VMEM((B,), jnp.int32),
            pltpu.VMEM((B,), jnp.int32),
            pltpu.VMEM((B,), jnp.int32),
            pltpu.VMEM((300,), jnp.float32),
            pltpu.VMEM((300,), jnp.float32),
            pltpu.VMEM((150,), jnp.float32),
            pltpu.VMEM((150,), jnp.float32),
            pltpu.VMEM((6, B), jnp.float32),
            pltpu.VMEM((B,), jnp.int32),
        ],
    )
    def lookup(cam_h, id_h, dt_h, wc1_h, bc1_h, wc2_h, bc2_h,
               wb_out, rows_out,
               cam_v, id_v, dt_v, wc1_v, bc1_v, wc2_v, bc2_v,
               wb_v, rows_v):
        wid = lax.axis_index("s") * 2 + lax.axis_index("c")

        @pl.when(wid == 0)
        def _():
            pltpu.sync_copy(cam_h, cam_v)
            pltpu.sync_copy(id_h, id_v)
            pltpu.sync_copy(dt_h, dt_v)
            pltpu.sync_copy(wc1_h, wc1_v)
            pltpu.sync_copy(bc1_h, bc1_v)
            pltpu.sync_copy(wc2_h, wc2_v)
            pltpu.sync_copy(bc2_h, bc2_v)
            cam3 = cam_v[...] * 3
            use1 = dt_v[...] == 0
            rows_v[...] = jnp.where(use1, id_v[...], NIDENT1 + id_v[...])
            for c in range(3):
                wb_v[c, :] = jnp.where(
                    use1,
                    plsc.load_gather(wc1_v, [cam3 + c]),
                    plsc.load_gather(wc2_v, [cam3 + c]))
                wb_v[3 + c, :] = jnp.where(
                    use1,
                    plsc.load_gather(bc1_v, [cam3 + c]),
                    plsc.load_gather(bc2_v, [cam3 + c]))
            pltpu.sync_copy(wb_v, wb_out)
            pltpu.sync_copy(rows_v, rows_out)

    return lookup(camindex, idindex, dataset_type,
                  wcam1f, bcam1f, wcam2f, bcam2f)


def _affine_body(rows_ref, wbcam_ref, *refs):
    wi_refs = refs[0:NB]
    bi_refs = refs[NB:2 * NB]
    img_ref = refs[2 * NB]
    out_ref = refs[2 * NB + 1]
    b_i = pl.program_id(0)
    for j in range(NB):
        for c in range(3):
            s = b_i * NB + j
            w = wbcam_ref[c, s] + wi_refs[j][0, 0, c]
            bb = wbcam_ref[3 + c, s] + bi_refs[j][0, 0, c]
            out_ref[j, c] = img_ref[j, c] * w + bb


def _tc_affine(identrows, wbcam, wident_cat, bident_cat, image):
    def row_map(j):
        return lambda bi, rows: (rows[bi * NB + j], 0, 0)

    grid_spec = pltpu.PrefetchScalarGridSpec(
        num_scalar_prefetch=1,
        grid=(B // NB,),
        in_specs=[
            pl.BlockSpec(memory_space=pltpu.SMEM),
        ] + [
            pl.BlockSpec((1, 1, 3), row_map(j)) for j in range(NB)
        ] + [
            pl.BlockSpec((1, 1, 3), row_map(j)) for j in range(NB)
        ] + [
            pl.BlockSpec((NB, 3, 512, 512), lambda bi, rows: (bi, 0, 0, 0)),
        ],
        out_specs=pl.BlockSpec((NB, 3, 512, 512),
                               lambda bi, rows: (bi, 0, 0, 0)),
    )
    wi3 = wident_cat.reshape(-1, 1, 3)
    bi3 = bident_cat.reshape(-1, 1, 3)
    return pl.pallas_call(
        _affine_body,
        grid_spec=grid_spec,
        out_shape=jax.ShapeDtypeStruct(image.shape, image.dtype),
        compiler_params=pltpu.CompilerParams(
            dimension_semantics=("arbitrary",)),
    )(identrows, wbcam, *([wi3] * NB), *([bi3] * NB), image)


@jax.jit
def kernel(image, camindex, idindex, dataset_type,
           wcam1, bcam1, wident1, bident1,
           wcam2, bcam2, wident2, bident2):
    wbcam, identrows = _sc_lookup(
        camindex, idindex, dataset_type,
        wcam1.reshape(-1), bcam1.reshape(-1),
        wcam2.reshape(-1), bcam2.reshape(-1))
    wident_cat = jnp.concatenate([wident1, wident2])
    bident_cat = jnp.concatenate([bident1, bident2])
    return _tc_affine(identrows, wbcam, wident_cat, bident_cat, image)
